# Initial kernel scaffold; baseline (speedup 1.0000x reference)
import jax
import jax.numpy as jnp
from jax.experimental import pallas as pl


def kernel(x, edge_index, nW1, nb1, nW2, nb2, c0W1, c0b1, c0W2, c0b2, c1W1, c1b1, c1W2, c1b2, c2W1, c2b1, c2W2, c2b2):
    raise NotImplementedError("write your pallas kernel here")


# speedup vs baseline: 1.4890x; 1.4890x over previous
import functools

import jax
import jax.numpy as jnp
from jax import lax
from jax.experimental import pallas as pl
from jax.experimental.pallas import tpu as pltpu
from jax.experimental.pallas import tpu_sc as plsc

N = 10000
E = 320000
D_IN = 128
H = 64

NC = 2
NS = 16
NW = NC * NS

NPT2 = 160
NPAD2 = NW * NPT2
EPT = E // NW
GCH = 400
RCH = 8000
CAP = 16384
BCH = 256


def _enc_body(x_ref, w1_ref, b1_ref, w2_ref, b2_ref, o_ref):
    h1 = jnp.maximum(
        jnp.dot(x_ref[...], w1_ref[...], preferred_element_type=jnp.float32,
                precision=lax.Precision.HIGHEST)
        + b1_ref[...], 0.0)
    o_ref[...] = (
        jnp.dot(h1, w2_ref[...], preferred_element_type=jnp.float32,
                precision=lax.Precision.HIGHEST)
        + b2_ref[...])


def _encode(x, w1, b1, w2, b2):
    blk = 2000
    return pl.pallas_call(
        _enc_body,
        grid=(N // blk,),
        in_specs=[
            pl.BlockSpec((blk, D_IN), lambda i: (i, 0)),
            pl.BlockSpec((D_IN, H), lambda i: (0, 0)),
            pl.BlockSpec((1, H), lambda i: (0, 0)),
            pl.BlockSpec((H, H), lambda i: (0, 0)),
            pl.BlockSpec((1, H), lambda i: (0, 0)),
        ],
        out_specs=pl.BlockSpec((blk, H), lambda i: (i, 0)),
        out_shape=jax.ShapeDtypeStruct((N, H), jnp.float32),
    )(x, w1, b1.reshape(1, H), w2, b2.reshape(1, H))


def _ab_body(h_ref, w1_ref, b1_ref, o_ref):
    h = h_ref[...]
    w1 = w1_ref[...]
    wd = w1[0:H, :] - w1[H:2 * H, :]
    ws = w1[H:2 * H, :]
    a = jnp.dot(h, wd, preferred_element_type=jnp.float32,
                precision=lax.Precision.HIGHEST) + b1_ref[...]
    b = jnp.dot(h, ws, preferred_element_type=jnp.float32,
                precision=lax.Precision.HIGHEST)
    o_ref[...] = jnp.concatenate([a, b], axis=1)


def _node_tables(h, w1, b1):
    blk = 2000
    return pl.pallas_call(
        _ab_body,
        grid=(N // blk,),
        in_specs=[
            pl.BlockSpec((blk, H), lambda i: (i, 0)),
            pl.BlockSpec((2 * H, H), lambda i: (0, 0)),
            pl.BlockSpec((1, H), lambda i: (0, 0)),
        ],
        out_specs=pl.BlockSpec((blk, 2 * H), lambda i: (i, 0)),
        out_shape=jax.ShapeDtypeStruct((N, 2 * H), jnp.float32),
    )(h, w1, b1.reshape(1, H))


def _msg_body(t_ref, w2_ref, b2_ref, o_ref):
    t = t_ref[...]
    w2 = w2_ref[...]
    b2 = b2_ref[...]
    ma = jnp.maximum(
        jnp.dot(t[:, :H], w2, preferred_element_type=jnp.float32,
                precision=lax.Precision.HIGHEST) + b2, 0.0)
    mb = jnp.maximum(
        jnp.dot(t[:, H:], w2, preferred_element_type=jnp.float32,
                precision=lax.Precision.HIGHEST) + b2, 0.0)
    o_ref[...] = jnp.concatenate([ma, mb], axis=1)


def _messages(t, w2, b2):
    blk2 = 1000
    return pl.pallas_call(
        _msg_body,
        grid=(E // 2 // blk2,),
        in_specs=[
            pl.BlockSpec((blk2, 2 * H), lambda i: (i, 0)),
            pl.BlockSpec((H, H), lambda i: (0, 0)),
            pl.BlockSpec((1, H), lambda i: (0, 0)),
        ],
        out_specs=pl.BlockSpec((blk2, 2 * H), lambda i: (i, 0)),
        out_shape=jax.ShapeDtypeStruct((E // 2, 2 * H), jnp.float32),
    )(t, w2, b2.reshape(1, H))


def _gather_body(ei, ab_hbm, t_hbm, sidx, didx, bufs, bufd, buft, sem1, sem2):
    wid = lax.axis_index("s") * NC + lax.axis_index("c")
    base = wid * EPT

    def chunk(ck, carry):
        off = pl.multiple_of(base + ck * GCH, GCH)
        pltpu.sync_copy(ei.at[pl.ds(off, GCH)], sidx)
        pltpu.sync_copy(ei.at[pl.ds(E + off, GCH)], didx)
        ca = pltpu.async_copy(ab_hbm.at[didx], bufd, sem1)
        cb = pltpu.async_copy(ab_hbm.at[sidx], bufs, sem2)
        ca.wait()
        cb.wait()

        def row(k, c2):
            for c in range(H // 16):
                sa = pl.ds(c * 16, 16)
                sb = pl.ds(H + c * 16, 16)
                buft[k, sa] = jnp.maximum(bufd[2 * k, sa] + bufs[2 * k, sb],
                                          0.0)
                buft[k, sb] = jnp.maximum(
                    bufd[2 * k + 1, sa] + bufs[2 * k + 1, sb], 0.0)
            return c2

        lax.fori_loop(0, GCH // 2, row, 0)
        pltpu.sync_copy(buft,
                        t_hbm.at[pl.ds(pl.multiple_of(off // 2, GCH // 2),
                                       GCH // 2)])
        return carry

    lax.fori_loop(0, EPT // GCH, chunk, 0)


def _gather_combine(edge_index, ab):
    mesh = plsc.VectorSubcoreMesh(core_axis_name="c", subcore_axis_name="s")
    f = functools.partial(
        pl.kernel,
        out_type=jax.ShapeDtypeStruct((E // 2, 2 * H), jnp.float32),
        mesh=mesh,
        scratch_types=[
            pltpu.VMEM((GCH,), jnp.int32),
            pltpu.VMEM((GCH,), jnp.int32),
            pltpu.VMEM((GCH, 2 * H), jnp.float32),
            pltpu.VMEM((GCH, 2 * H), jnp.float32),
            pltpu.VMEM((GCH // 2, 2 * H), jnp.float32),
            pltpu.SemaphoreType.DMA,
            pltpu.SemaphoreType.DMA,
        ],
    )(_gather_body)
    return f(edge_index, ab)


def _scatter_body(ei, m_hbm, out_hbm, sbuf, dlist, rowlist, mbuf, acc, cntbuf,
                  sem):
    wid = lax.axis_index("s") * NC + lax.axis_index("c")
    lo2 = wid * NPT2
    iota = lax.iota(jnp.int32, 16)

    pad_code = jnp.full((16,), (lo2 + NPT2) * 4, jnp.int32)
    zeros_i = jnp.zeros((16,), jnp.int32)
    zeros_f = jnp.zeros((16,), jnp.float32)

    def pre_lists(i, c2):
        dlist[pl.ds(i * 16, 16)] = pad_code
        rowlist[pl.ds(i * 16, 16)] = zeros_i
        return c2

    lax.fori_loop(0, (CAP + 16) // 16, pre_lists, 0)

    def pre_acc(i, c2):
        for c in range(2 * H // 16):
            acc[i, pl.ds(c * 16, 16)] = zeros_f
        return c2

    lax.fori_loop(0, NPT2 + 1, pre_acc, 0)

    def scan_chunk(ck, ptrv):
        pltpu.sync_copy(
            ei.at[pl.ds(pl.multiple_of(E + ck * RCH, RCH), RCH)], sbuf)

        def scan_v(v, ptrv):
            d = sbuf[pl.ds(v * 16, 16)]
            pr = d >> 1
            mask = (pr >= lo2) & (pr < lo2 + NPT2)
            eid = ck * RCH + v * 16 + iota
            keys = jnp.where(mask, 0, 16) + iota
            _, sd = plsc.sort_key_val(keys,
                                      jnp.where(mask, d * 2 + (eid & 1),
                                                pad_code))
            _, sr = plsc.sort_key_val(keys, jnp.where(mask, eid >> 1, 0))
            offs = ptrv + iota
            plsc.store_scatter(dlist, [offs], sd)
            plsc.store_scatter(rowlist, [offs], sr)
            return ptrv + plsc.all_reduce_population_count(mask)

        return lax.fori_loop(0, RCH // 16, scan_v, ptrv)

    cntv = lax.fori_loop(0, E // RCH, scan_chunk,
                         jnp.zeros((16,), jnp.int32))

    cnt = cntv[0]
    trips = (cnt + (BCH - 1)) // BCH

    def rmw_chunk(ck, c2):
        boff = pl.multiple_of(ck * BCH, BCH)
        pltpu.async_copy(m_hbm.at[rowlist.at[pl.ds(boff, BCH)]], mbuf,
                         sem).wait()

        def edge(j, c3):
            pos = ck * BCH + j
            dcode = plsc.load_gather(dlist, [jnp.full((16,), pos, jnp.int32)])
            d = dcode >> 1
            rows = (d >> 1) - lo2
            jrow = jnp.full((16,), j, jnp.int32)
            for c in range(H // 16):
                acols = (d & 1) * H + c * 16 + iota
                mcols = (dcode & 1) * H + c * 16 + iota
                mv = plsc.load_gather(mbuf, [jrow, mcols])
                av = plsc.load_gather(acc, [rows, acols])
                plsc.store_scatter(acc, [rows, acols], jnp.maximum(av, mv))
            return c3

        lax.fori_loop(0, BCH, edge, 0)
        return c2

    lax.fori_loop(0, trips, rmw_chunk, 0)

    pltpu.sync_copy(acc.at[pl.ds(0, NPT2)],
                    out_hbm.at[pl.ds(pl.multiple_of(lo2, NPT2), NPT2)])


def _segment_max(edge_index, m):
    mesh = plsc.VectorSubcoreMesh(core_axis_name="c", subcore_axis_name="s")
    f = functools.partial(
        pl.kernel,
        out_type=jax.ShapeDtypeStruct((NPAD2, 2 * H), jnp.float32),
        mesh=mesh,
        compiler_params=pltpu.CompilerParams(needs_layout_passes=False),
        scratch_types=[
            pltpu.VMEM((RCH,), jnp.int32),
            pltpu.VMEM((CAP + 16,), jnp.int32),
            pltpu.VMEM((CAP + 16,), jnp.int32),
            pltpu.VMEM((BCH, 2 * H), jnp.float32),
            pltpu.VMEM((NPT2 + 1, 2 * H), jnp.float32),
            pltpu.VMEM((16,), jnp.int32),
            pltpu.SemaphoreType.DMA,
        ],
    )(_scatter_body)
    return f(edge_index, m)


def kernel(x, edge_index, nW1, nb1, nW2, nb2,
           c0W1, c0b1, c0W2, c0b2,
           c1W1, c1b1, c1W2, c1b2,
           c2W1, c2b1, c2W2, c2b2):
    ei_flat = edge_index.reshape(2 * E)
    h = _encode(x, nW1, nb1, nW2, nb2)
    for (w1, b1, w2, b2) in ((c0W1, c0b1, c0W2, c0b2),
                             (c1W1, c1b1, c1W2, c1b2),
                             (c2W1, c2b1, c2W2, c2b2)):
        ab = _node_tables(h, w1, b1)
        t = _gather_combine(ei_flat, ab)
        m = _messages(t, w2, b2)
        h = _segment_max(ei_flat, m).reshape(2 * NPAD2, H)[:N]
    return h


# speedup vs baseline: 2.1033x; 1.4126x over previous
import functools

import jax
import jax.numpy as jnp
from jax import lax
from jax.experimental import pallas as pl
from jax.experimental.pallas import tpu as pltpu
from jax.experimental.pallas import tpu_sc as plsc

N = 10000
E = 320000
D_IN = 128
H = 64

NC = 2
NS = 16
NW = NC * NS

NPT2 = 160
NPAD2 = NW * NPT2
EPT = E // NW
GCH = 80
RCH = 8000
CAP = 16384
BCH = 256


def _enc_body(x_ref, w1_ref, b1_ref, w2_ref, b2_ref, o_ref):
    h1 = jnp.maximum(
        jnp.dot(x_ref[...], w1_ref[...], preferred_element_type=jnp.float32,
                precision=lax.Precision.HIGHEST)
        + b1_ref[...], 0.0)
    o_ref[...] = (
        jnp.dot(h1, w2_ref[...], preferred_element_type=jnp.float32,
                precision=lax.Precision.HIGHEST)
        + b2_ref[...])


def _encode(x, w1, b1, w2, b2):
    blk = 2000
    return pl.pallas_call(
        _enc_body,
        grid=(N // blk,),
        in_specs=[
            pl.BlockSpec((blk, D_IN), lambda i: (i, 0)),
            pl.BlockSpec((D_IN, H), lambda i: (0, 0)),
            pl.BlockSpec((1, H), lambda i: (0, 0)),
            pl.BlockSpec((H, H), lambda i: (0, 0)),
            pl.BlockSpec((1, H), lambda i: (0, 0)),
        ],
        out_specs=pl.BlockSpec((blk, H), lambda i: (i, 0)),
        out_shape=jax.ShapeDtypeStruct((N, H), jnp.float32),
    )(x, w1, b1.reshape(1, H), w2, b2.reshape(1, H))


def _ab_body(h_ref, w1_ref, b1_ref, o_ref):
    h = h_ref[...]
    w1 = w1_ref[...]
    wd = w1[0:H, :] - w1[H:2 * H, :]
    ws = w1[H:2 * H, :]
    a = jnp.dot(h, wd, preferred_element_type=jnp.float32,
                precision=lax.Precision.HIGHEST) + b1_ref[...]
    b = jnp.dot(h, ws, preferred_element_type=jnp.float32,
                precision=lax.Precision.HIGHEST)
    o_ref[...] = jnp.concatenate([a, b], axis=1)


def _node_tables(h, w1, b1):
    blk = 2000
    return pl.pallas_call(
        _ab_body,
        grid=(N // blk,),
        in_specs=[
            pl.BlockSpec((blk, H), lambda i: (i, 0)),
            pl.BlockSpec((2 * H, H), lambda i: (0, 0)),
            pl.BlockSpec((1, H), lambda i: (0, 0)),
        ],
        out_specs=pl.BlockSpec((blk, 2 * H), lambda i: (i, 0)),
        out_shape=jax.ShapeDtypeStruct((N, 2 * H), jnp.float32),
    )(h, w1, b1.reshape(1, H))


def _msg_body(t_ref, w2_ref, b2_ref, o_ref):
    t = t_ref[...]
    w2 = w2_ref[...]
    b2 = b2_ref[...]
    ma = jnp.maximum(
        jnp.dot(t[:, :H], w2, preferred_element_type=jnp.float32,
                precision=lax.Precision.HIGHEST) + b2, 0.0)
    mb = jnp.maximum(
        jnp.dot(t[:, H:], w2, preferred_element_type=jnp.float32,
                precision=lax.Precision.HIGHEST) + b2, 0.0)
    o_ref[...] = jnp.concatenate([ma, mb], axis=1)


def _messages(t, w2, b2):
    blk2 = 1000
    return pl.pallas_call(
        _msg_body,
        grid=(E // 2 // blk2,),
        in_specs=[
            pl.BlockSpec((blk2, 2 * H), lambda i: (i, 0)),
            pl.BlockSpec((H, H), lambda i: (0, 0)),
            pl.BlockSpec((1, H), lambda i: (0, 0)),
        ],
        out_specs=pl.BlockSpec((blk2, 2 * H), lambda i: (i, 0)),
        out_shape=jax.ShapeDtypeStruct((E // 2, 2 * H), jnp.float32),
    )(t, w2, b2.reshape(1, H))


def _gather_body(ei, ab_hbm, t_hbm, sidx, didx, bufs, bufd, buft,
                 semas, sembs, semt):
    wid = lax.axis_index("s") * NC + lax.axis_index("c")
    base = pl.multiple_of(wid * EPT, EPT)
    nch = EPT // GCH

    pltpu.sync_copy(ei.at[pl.ds(base, EPT)], sidx)
    pltpu.sync_copy(ei.at[pl.ds(E + base, EPT)], didx)

    def issue(ck, b):
        boff = pl.multiple_of(ck * GCH, GCH)
        pltpu.async_copy(ab_hbm.at[didx.at[pl.ds(boff, GCH)]], bufd[b],
                         semas[b])
        pltpu.async_copy(ab_hbm.at[sidx.at[pl.ds(boff, GCH)]], bufs[b],
                         sembs[b])

    def consume(ck, b, first):
        pltpu.make_async_copy(ab_hbm.at[didx.at[pl.ds(0, GCH)]], bufd[b],
                              semas[b]).wait()
        pltpu.make_async_copy(ab_hbm.at[sidx.at[pl.ds(0, GCH)]], bufs[b],
                              sembs[b]).wait()
        off2 = pl.multiple_of(base // 2 + ck * (GCH // 2), GCH // 2)
        if not first:
            pltpu.make_async_copy(buft, t_hbm.at[pl.ds(0, GCH // 2)],
                                  semt).wait()

        def row(k, c2):
            for c in range(H // 16):
                sa = pl.ds(c * 16, 16)
                sb = pl.ds(H + c * 16, 16)
                buft[k, sa] = jnp.maximum(
                    bufd[b][2 * k, sa] + bufs[b][2 * k, sb], 0.0)
                buft[k, sb] = jnp.maximum(
                    bufd[b][2 * k + 1, sa] + bufs[b][2 * k + 1, sb], 0.0)
            return c2

        lax.fori_loop(0, GCH // 2, row, 0, unroll=4)
        pltpu.async_copy(buft, t_hbm.at[pl.ds(off2, GCH // 2)], semt)

    issue(0, 0)
    issue(1, 1)
    consume(0, 0, True)

    def pair(j, carry):
        issue(2 * j + 2, 0)
        consume(2 * j + 1, 1, False)
        issue(2 * j + 3, 1)
        consume(2 * j + 2, 0, False)
        return carry

    lax.fori_loop(0, (nch - 3) // 2, pair, 0)
    issue(nch - 1, 0)
    consume(nch - 2, 1, False)
    consume(nch - 1, 0, False)
    pltpu.make_async_copy(buft, t_hbm.at[pl.ds(0, GCH // 2)], semt).wait()


def _gather_combine(edge_index, ab):
    mesh = plsc.VectorSubcoreMesh(core_axis_name="c", subcore_axis_name="s")
    f = functools.partial(
        pl.kernel,
        out_type=jax.ShapeDtypeStruct((E // 2, 2 * H), jnp.float32),
        mesh=mesh,
        scratch_types=[
            pltpu.VMEM((EPT,), jnp.int32),
            pltpu.VMEM((EPT,), jnp.int32),
            [pltpu.VMEM((GCH, 2 * H), jnp.float32)] * 2,
            [pltpu.VMEM((GCH, 2 * H), jnp.float32)] * 2,
            pltpu.VMEM((GCH // 2, 2 * H), jnp.float32),
            [pltpu.SemaphoreType.DMA] * 2,
            [pltpu.SemaphoreType.DMA] * 2,
            pltpu.SemaphoreType.DMA,
        ],
    )(_gather_body)
    return f(edge_index, ab)


def _route_body(ei, dl_hbm, rl_hbm, cnt_hbm, sbuf, dlist, rowlist, cbuf):
    wid = lax.axis_index("s") * NC + lax.axis_index("c")
    lo2 = wid * NPT2
    iota = lax.iota(jnp.int32, 16)

    pad_code = jnp.full((16,), (lo2 + NPT2) * 4, jnp.int32)
    zeros_i = jnp.zeros((16,), jnp.int32)

    def pre_lists(i, c2):
        dlist[pl.ds(i * 16, 16)] = pad_code
        rowlist[pl.ds(i * 16, 16)] = zeros_i
        return c2

    lax.fori_loop(0, (CAP + 16) // 16, pre_lists, 0, unroll=4)

    def scan_chunk(ck, ptrv):
        pltpu.sync_copy(
            ei.at[pl.ds(pl.multiple_of(E + ck * RCH, RCH), RCH)], sbuf)

        def scan_v(v, ptrv):
            d = sbuf[pl.ds(v * 16, 16)]
            pr = d >> 1
            mask = (pr >= lo2) & (pr < lo2 + NPT2)
            eid = ck * RCH + v * 16 + iota
            keys = jnp.where(mask, 0, 16) + iota
            _, sd = plsc.sort_key_val(keys,
                                      jnp.where(mask, d * 2 + (eid & 1),
                                                pad_code))
            _, sr = plsc.sort_key_val(keys, jnp.where(mask, eid >> 1, 0))
            offs = ptrv + iota
            plsc.store_scatter(dlist, [offs], sd)
            plsc.store_scatter(rowlist, [offs], sr)
            return ptrv + plsc.all_reduce_population_count(mask)

        return lax.fori_loop(0, RCH // 16, scan_v, ptrv, unroll=2)

    cntv = lax.fori_loop(0, E // RCH, scan_chunk,
                         jnp.zeros((16,), jnp.int32))

    cbuf[pl.ds(0, 16)] = cntv
    pltpu.sync_copy(dlist.at[pl.ds(0, CAP)],
                    dl_hbm.at[pl.ds(pl.multiple_of(wid * CAP, CAP), CAP)])
    pltpu.sync_copy(rowlist.at[pl.ds(0, CAP)],
                    rl_hbm.at[pl.ds(pl.multiple_of(wid * CAP, CAP), CAP)])
    pltpu.sync_copy(cbuf, cnt_hbm.at[pl.ds(pl.multiple_of(wid * 16, 16), 16)])


def _route(edge_index):
    mesh = plsc.VectorSubcoreMesh(core_axis_name="c", subcore_axis_name="s")
    f = functools.partial(
        pl.kernel,
        out_type=[
            jax.ShapeDtypeStruct((NW * CAP,), jnp.int32),
            jax.ShapeDtypeStruct((NW * CAP,), jnp.int32),
            jax.ShapeDtypeStruct((NW * 16,), jnp.int32),
        ],
        mesh=mesh,
        compiler_params=pltpu.CompilerParams(needs_layout_passes=False),
        scratch_types=[
            pltpu.VMEM((RCH,), jnp.int32),
            pltpu.VMEM((CAP + 16,), jnp.int32),
            pltpu.VMEM((CAP + 16,), jnp.int32),
            pltpu.VMEM((16,), jnp.int32),
        ],
    )(_route_body)
    return f(edge_index)


def _scatter_body(dl_hbm, rl_hbm, cnt_hbm, m_hbm, out_hbm, dlist, rowlist,
                  mbuf, acc, cbuf, sem):
    wid = lax.axis_index("s") * NC + lax.axis_index("c")
    lo2 = wid * NPT2
    iota = lax.iota(jnp.int32, 16)
    zeros_f = jnp.zeros((16,), jnp.float32)

    pltpu.sync_copy(cnt_hbm.at[pl.ds(pl.multiple_of(wid * 16, 16), 16)], cbuf)
    pltpu.sync_copy(dl_hbm.at[pl.ds(pl.multiple_of(wid * CAP, CAP), CAP)],
                    dlist)
    pltpu.sync_copy(rl_hbm.at[pl.ds(pl.multiple_of(wid * CAP, CAP), CAP)],
                    rowlist)

    def pre_acc(i, c2):
        for c in range(2 * H // 16):
            acc[i, pl.ds(c * 16, 16)] = zeros_f
        return c2

    lax.fori_loop(0, NPT2 + 1, pre_acc, 0, unroll=4)

    cnt = cbuf[pl.ds(0, 16)][0]
    trips = (cnt + (BCH - 1)) // BCH

    def rmw_chunk(ck, c2):
        boff = pl.multiple_of(ck * BCH, BCH)
        pltpu.async_copy(m_hbm.at[rowlist.at[pl.ds(boff, BCH)]], mbuf,
                         sem).wait()

        def edge(j, c3):
            pos = ck * BCH + j
            dcode = plsc.load_gather(dlist, [jnp.full((16,), pos, jnp.int32)])
            d = dcode >> 1
            rows = (d >> 1) - lo2
            jrow = jnp.full((16,), j, jnp.int32)
            for c in range(H // 16):
                acols = (d & 1) * H + c * 16 + iota
                mcols = (dcode & 1) * H + c * 16 + iota
                mv = plsc.load_gather(mbuf, [jrow, mcols])
                av = plsc.load_gather(acc, [rows, acols])
                plsc.store_scatter(acc, [rows, acols], jnp.maximum(av, mv))
            return c3

        lax.fori_loop(0, BCH, edge, 0)
        return c2

    lax.fori_loop(0, trips, rmw_chunk, 0)

    pltpu.sync_copy(acc.at[pl.ds(0, NPT2)],
                    out_hbm.at[pl.ds(pl.multiple_of(lo2, NPT2), NPT2)])


def _segment_max(dl, rl, cnts, m):
    mesh = plsc.VectorSubcoreMesh(core_axis_name="c", subcore_axis_name="s")
    f = functools.partial(
        pl.kernel,
        out_type=jax.ShapeDtypeStruct((NPAD2, 2 * H), jnp.float32),
        mesh=mesh,
        compiler_params=pltpu.CompilerParams(needs_layout_passes=False),
        scratch_types=[
            pltpu.VMEM((CAP,), jnp.int32),
            pltpu.VMEM((CAP,), jnp.int32),
            pltpu.VMEM((BCH, 2 * H), jnp.float32),
            pltpu.VMEM((NPT2 + 1, 2 * H), jnp.float32),
            pltpu.VMEM((16,), jnp.int32),
            pltpu.SemaphoreType.DMA,
        ],
    )(_scatter_body)
    return f(dl, rl, cnts, m)


def kernel(x, edge_index, nW1, nb1, nW2, nb2,
           c0W1, c0b1, c0W2, c0b2,
           c1W1, c1b1, c1W2, c1b2,
           c2W1, c2b1, c2W2, c2b2):
    ei_flat = edge_index.reshape(2 * E)
    h = _encode(x, nW1, nb1, nW2, nb2)
    dl, rl, cnts = _route(ei_flat)
    for (w1, b1, w2, b2) in ((c0W1, c0b1, c0W2, c0b2),
                             (c1W1, c1b1, c1W2, c1b2),
                             (c2W1, c2b1, c2W2, c2b2)):
        ab = _node_tables(h, w1, b1)
        t = _gather_combine(ei_flat, ab)
        m = _messages(t, w2, b2)
        h = _segment_max(dl, rl, cnts, m).reshape(2 * NPAD2, H)[:N]
    return h


# speedup vs baseline: 2.5136x; 1.1951x over previous
import functools

import jax
import jax.numpy as jnp
from jax import lax
from jax.experimental import pallas as pl
from jax.experimental.pallas import tpu as pltpu
from jax.experimental.pallas import tpu_sc as plsc

N = 10000
E = 320000
D_IN = 128
H = 64

NC = 2
NS = 16
NW = NC * NS

NPT2 = 160
NPAD2 = NW * NPT2
EPT = E // NW
GCH = 80
RCH = 8000
CAP = 16384
BCH = 256


def _enc_body(x_ref, w1_ref, b1_ref, w2_ref, b2_ref, o_ref):
    h1 = jnp.maximum(
        jnp.dot(x_ref[...], w1_ref[...], preferred_element_type=jnp.float32,
                precision=lax.Precision.HIGHEST)
        + b1_ref[...], 0.0)
    o_ref[...] = (
        jnp.dot(h1, w2_ref[...], preferred_element_type=jnp.float32,
                precision=lax.Precision.HIGHEST)
        + b2_ref[...])


def _encode(x, w1, b1, w2, b2):
    blk = 2000
    return pl.pallas_call(
        _enc_body,
        grid=(N // blk,),
        in_specs=[
            pl.BlockSpec((blk, D_IN), lambda i: (i, 0)),
            pl.BlockSpec((D_IN, H), lambda i: (0, 0)),
            pl.BlockSpec((1, H), lambda i: (0, 0)),
            pl.BlockSpec((H, H), lambda i: (0, 0)),
            pl.BlockSpec((1, H), lambda i: (0, 0)),
        ],
        out_specs=pl.BlockSpec((blk, H), lambda i: (i, 0)),
        out_shape=jax.ShapeDtypeStruct((N, H), jnp.float32),
    )(x, w1, b1.reshape(1, H), w2, b2.reshape(1, H))


def _ab_body(h_ref, w1_ref, b1_ref, o_ref):
    h = h_ref[...]
    w1 = w1_ref[...]
    wd = w1[0:H, :] - w1[H:2 * H, :]
    ws = w1[H:2 * H, :]
    a = jnp.dot(h, wd, preferred_element_type=jnp.float32,
                precision=lax.Precision.HIGHEST) + b1_ref[...]
    b = jnp.dot(h, ws, preferred_element_type=jnp.float32,
                precision=lax.Precision.HIGHEST)
    o_ref[...] = jnp.concatenate([a, b], axis=1)


def _node_tables(h, w1, b1):
    blk = 2000
    return pl.pallas_call(
        _ab_body,
        grid=(N // blk,),
        in_specs=[
            pl.BlockSpec((blk, H), lambda i: (i, 0)),
            pl.BlockSpec((2 * H, H), lambda i: (0, 0)),
            pl.BlockSpec((1, H), lambda i: (0, 0)),
        ],
        out_specs=pl.BlockSpec((blk, 2 * H), lambda i: (i, 0)),
        out_shape=jax.ShapeDtypeStruct((N, 2 * H), jnp.float32),
    )(h, w1, b1.reshape(1, H))


def _msg_body(t_ref, w2_ref, b2_ref, o_ref):
    t = t_ref[...]
    w2 = w2_ref[...]
    b2 = b2_ref[...]
    ma = jnp.maximum(
        jnp.dot(t[:, :H], w2, preferred_element_type=jnp.float32,
                precision=lax.Precision.HIGHEST) + b2, 0.0)
    mb = jnp.maximum(
        jnp.dot(t[:, H:], w2, preferred_element_type=jnp.float32,
                precision=lax.Precision.HIGHEST) + b2, 0.0)
    o_ref[...] = jnp.concatenate([ma, mb], axis=1)


def _messages(t, w2, b2):
    blk2 = 1000
    return pl.pallas_call(
        _msg_body,
        grid=(E // 2 // blk2,),
        in_specs=[
            pl.BlockSpec((blk2, 2 * H), lambda i: (i, 0)),
            pl.BlockSpec((H, H), lambda i: (0, 0)),
            pl.BlockSpec((1, H), lambda i: (0, 0)),
        ],
        out_specs=pl.BlockSpec((blk2, 2 * H), lambda i: (i, 0)),
        out_shape=jax.ShapeDtypeStruct((E // 2, 2 * H), jnp.float32),
    )(t, w2, b2.reshape(1, H))


def _gather_body(ei, ab_hbm, t_hbm, sidx, didx, bufs, bufd, buft,
                 semas, sembs, semt):
    wid = lax.axis_index("s") * NC + lax.axis_index("c")
    base = pl.multiple_of(wid * EPT, EPT)
    nch = EPT // GCH

    pltpu.sync_copy(ei.at[pl.ds(base, EPT)], sidx)
    pltpu.sync_copy(ei.at[pl.ds(E + base, EPT)], didx)

    def issue(ck, b):
        boff = pl.multiple_of(ck * GCH, GCH)
        pltpu.async_copy(ab_hbm.at[didx.at[pl.ds(boff, GCH)]], bufd[b],
                         semas[b])
        pltpu.async_copy(ab_hbm.at[sidx.at[pl.ds(boff, GCH)]], bufs[b],
                         sembs[b])

    def consume(ck, b, first):
        pltpu.make_async_copy(ab_hbm.at[didx.at[pl.ds(0, GCH)]], bufd[b],
                              semas[b]).wait()
        pltpu.make_async_copy(ab_hbm.at[sidx.at[pl.ds(0, GCH)]], bufs[b],
                              sembs[b]).wait()
        off2 = pl.multiple_of(base // 2 + ck * (GCH // 2), GCH // 2)
        if not first:
            pltpu.make_async_copy(buft, t_hbm.at[pl.ds(0, GCH // 2)],
                                  semt).wait()

        def row(k, c2):
            for c in range(H // 16):
                sa = pl.ds(c * 16, 16)
                sb = pl.ds(H + c * 16, 16)
                buft[k, sa] = jnp.maximum(
                    bufd[b][2 * k, sa] + bufs[b][2 * k, sb], 0.0)
                buft[k, sb] = jnp.maximum(
                    bufd[b][2 * k + 1, sa] + bufs[b][2 * k + 1, sb], 0.0)
            return c2

        lax.fori_loop(0, GCH // 2, row, 0, unroll=4)
        pltpu.async_copy(buft, t_hbm.at[pl.ds(off2, GCH // 2)], semt)

    issue(0, 0)
    issue(1, 1)
    consume(0, 0, True)

    def pair(j, carry):
        issue(2 * j + 2, 0)
        consume(2 * j + 1, 1, False)
        issue(2 * j + 3, 1)
        consume(2 * j + 2, 0, False)
        return carry

    lax.fori_loop(0, (nch - 3) // 2, pair, 0)
    issue(nch - 1, 0)
    consume(nch - 2, 1, False)
    consume(nch - 1, 0, False)
    pltpu.make_async_copy(buft, t_hbm.at[pl.ds(0, GCH // 2)], semt).wait()


def _gather_combine(edge_index, ab):
    mesh = plsc.VectorSubcoreMesh(core_axis_name="c", subcore_axis_name="s")
    f = functools.partial(
        pl.kernel,
        out_type=jax.ShapeDtypeStruct((E // 2, 2 * H), jnp.float32),
        mesh=mesh,
        scratch_types=[
            pltpu.VMEM((EPT,), jnp.int32),
            pltpu.VMEM((EPT,), jnp.int32),
            [pltpu.VMEM((GCH, 2 * H), jnp.float32)] * 2,
            [pltpu.VMEM((GCH, 2 * H), jnp.float32)] * 2,
            pltpu.VMEM((GCH // 2, 2 * H), jnp.float32),
            [pltpu.SemaphoreType.DMA] * 2,
            [pltpu.SemaphoreType.DMA] * 2,
            pltpu.SemaphoreType.DMA,
        ],
    )(_gather_body)
    return f(edge_index, ab)


def _route_body(ei, dl_hbm, rl_hbm, cnt_hbm, sbuf, dlist, rowlist, cbuf):
    wid = lax.axis_index("s") * NC + lax.axis_index("c")
    lo2 = wid * NPT2
    iota = lax.iota(jnp.int32, 16)

    pad_code = jnp.full((16,), (lo2 + NPT2) * 4, jnp.int32)
    zeros_i = jnp.zeros((16,), jnp.int32)

    def pre_lists(i, c2):
        dlist[pl.ds(i * 16, 16)] = pad_code
        rowlist[pl.ds(i * 16, 16)] = zeros_i
        return c2

    lax.fori_loop(0, (CAP + 16) // 16, pre_lists, 0, unroll=4)

    def scan_chunk(ck, ptrv):
        pltpu.sync_copy(
            ei.at[pl.ds(pl.multiple_of(E + ck * RCH, RCH), RCH)], sbuf)

        def scan_v(v, ptrv):
            d = sbuf[pl.ds(v * 16, 16)]
            pr = d >> 1
            mask = (pr >= lo2) & (pr < lo2 + NPT2)
            eid = ck * RCH + v * 16 + iota
            keys = jnp.where(mask, 0, 16) + iota
            _, sd = plsc.sort_key_val(keys,
                                      jnp.where(mask, d * 2 + (eid & 1),
                                                pad_code))
            _, sr = plsc.sort_key_val(keys, jnp.where(mask, eid >> 1, 0))
            offs = ptrv + iota
            plsc.store_scatter(dlist, [offs], sd)
            plsc.store_scatter(rowlist, [offs], sr)
            return ptrv + plsc.all_reduce_population_count(mask)

        return lax.fori_loop(0, RCH // 16, scan_v, ptrv, unroll=2)

    cntv = lax.fori_loop(0, E // RCH, scan_chunk,
                         jnp.zeros((16,), jnp.int32))

    cbuf[pl.ds(0, 16)] = cntv
    pltpu.sync_copy(dlist.at[pl.ds(0, CAP)],
                    dl_hbm.at[pl.ds(pl.multiple_of(wid * CAP, CAP), CAP)])
    pltpu.sync_copy(rowlist.at[pl.ds(0, CAP)],
                    rl_hbm.at[pl.ds(pl.multiple_of(wid * CAP, CAP), CAP)])
    pltpu.sync_copy(cbuf, cnt_hbm.at[pl.ds(pl.multiple_of(wid * 16, 16), 16)])


def _route(edge_index):
    mesh = plsc.VectorSubcoreMesh(core_axis_name="c", subcore_axis_name="s")
    f = functools.partial(
        pl.kernel,
        out_type=[
            jax.ShapeDtypeStruct((NW * CAP,), jnp.int32),
            jax.ShapeDtypeStruct((NW * CAP,), jnp.int32),
            jax.ShapeDtypeStruct((NW * 16,), jnp.int32),
        ],
        mesh=mesh,
        compiler_params=pltpu.CompilerParams(needs_layout_passes=False),
        scratch_types=[
            pltpu.VMEM((RCH,), jnp.int32),
            pltpu.VMEM((CAP + 16,), jnp.int32),
            pltpu.VMEM((CAP + 16,), jnp.int32),
            pltpu.VMEM((16,), jnp.int32),
        ],
    )(_route_body)
    return f(edge_index)


def _scatter_body(dl_hbm, rl_hbm, cnt_hbm, m_hbm, out_hbm, dlist, rowlist,
                  mbufs, acc, cbuf, sems):
    wid = lax.axis_index("s") * NC + lax.axis_index("c")
    lo2 = wid * NPT2
    zeros_f = jnp.zeros((16,), jnp.float32)

    pltpu.sync_copy(cnt_hbm.at[pl.ds(pl.multiple_of(wid * 16, 16), 16)], cbuf)
    pltpu.sync_copy(dl_hbm.at[pl.ds(pl.multiple_of(wid * CAP, CAP), CAP)],
                    dlist)
    pltpu.sync_copy(rl_hbm.at[pl.ds(pl.multiple_of(wid * CAP, CAP), CAP)],
                    rowlist)

    def pre_acc(i, c2):
        for c in range(2 * H // 16):
            acc[i, pl.ds(c * 16, 16)] = zeros_f
        return c2

    lax.fori_loop(0, NPT2 + 1, pre_acc, 0, unroll=4)

    cnt = cbuf[pl.ds(0, 16)][0]
    trips = (cnt + (BCH - 1)) // BCH

    def issue(ck, b):
        @pl.when(ck < trips)
        def _():
            boff = pl.multiple_of(ck * BCH, BCH)
            pltpu.async_copy(m_hbm.at[rowlist.at[pl.ds(boff, BCH)]],
                             mbufs[b], sems[b])

    def process(ck, b):
        @pl.when(ck < trips)
        def _():
            pltpu.make_async_copy(m_hbm.at[rowlist.at[pl.ds(0, BCH)]],
                                  mbufs[b], sems[b]).wait()

            def grp(g, c3):
                base16 = ck * BCH + g * 16
                dv = dlist[pl.ds(base16, 16)]
                for e in range(16):
                    dc = dv[e]
                    d = dc >> 1
                    r = (d >> 1) - lo2
                    ab_ = (d & 1) * H
                    mb_ = (dc & 1) * H
                    j = g * 16 + e
                    for c in range(H // 16):
                        mv = mbufs[b][j, pl.ds(mb_ + c * 16, 16)]
                        av = acc[r, pl.ds(ab_ + c * 16, 16)]
                        acc[r, pl.ds(ab_ + c * 16, 16)] = jnp.maximum(av, mv)
                return c3

            lax.fori_loop(0, BCH // 16, grp, 0)

    issue(0, 0)

    def pairb(j, c2):
        issue(2 * j + 1, 1)
        process(2 * j, 0)
        issue(2 * j + 2, 0)
        process(2 * j + 1, 1)
        return c2

    lax.fori_loop(0, (trips + 1) // 2, pairb, 0)

    pltpu.sync_copy(acc.at[pl.ds(0, NPT2)],
                    out_hbm.at[pl.ds(pl.multiple_of(lo2, NPT2), NPT2)])


def _segment_max(dl, rl, cnts, m):
    mesh = plsc.VectorSubcoreMesh(core_axis_name="c", subcore_axis_name="s")
    f = functools.partial(
        pl.kernel,
        out_type=jax.ShapeDtypeStruct((NPAD2, 2 * H), jnp.float32),
        mesh=mesh,
        compiler_params=pltpu.CompilerParams(needs_layout_passes=False),
        scratch_types=[
            pltpu.VMEM((CAP,), jnp.int32),
            pltpu.VMEM((CAP,), jnp.int32),
            [pltpu.VMEM((BCH, 2 * H), jnp.float32)] * 2,
            pltpu.VMEM((NPT2 + 1, 2 * H), jnp.float32),
            pltpu.VMEM((16,), jnp.int32),
            [pltpu.SemaphoreType.DMA] * 2,
        ],
    )(_scatter_body)
    return f(dl, rl, cnts, m)


def kernel(x, edge_index, nW1, nb1, nW2, nb2,
           c0W1, c0b1, c0W2, c0b2,
           c1W1, c1b1, c1W2, c1b2,
           c2W1, c2b1, c2W2, c2b2):
    ei_flat = edge_index.reshape(2 * E)
    h = _encode(x, nW1, nb1, nW2, nb2)
    dl, rl, cnts = _route(ei_flat)
    for (w1, b1, w2, b2) in ((c0W1, c0b1, c0W2, c0b2),
                             (c1W1, c1b1, c1W2, c1b2),
                             (c2W1, c2b1, c2W2, c2b2)):
        ab = _node_tables(h, w1, b1)
        t = _gather_combine(ei_flat, ab)
        m = _messages(t, w2, b2)
        h = _segment_max(dl, rl, cnts, m).reshape(2 * NPAD2, H)[:N]
    return h


# speedup vs baseline: 2.5249x; 1.0045x over previous
import functools

import jax
import jax.numpy as jnp
from jax import lax
from jax.experimental import pallas as pl
from jax.experimental.pallas import tpu as pltpu
from jax.experimental.pallas import tpu_sc as plsc

N = 10000
E = 320000
D_IN = 128
H = 64

NC = 2
NS = 16
NW = NC * NS

NPT2 = 160
NPAD2 = NW * NPT2
EPT = E // NW
GCH = 80
RCH = 8000
CAP = 16384
BCH = 256


def _enc_body(x_ref, w1_ref, b1_ref, w2_ref, b2_ref, o_ref):
    h1 = jnp.maximum(
        jnp.dot(x_ref[...], w1_ref[...], preferred_element_type=jnp.float32,
                precision=lax.Precision.HIGHEST)
        + b1_ref[...], 0.0)
    o_ref[...] = (
        jnp.dot(h1, w2_ref[...], preferred_element_type=jnp.float32,
                precision=lax.Precision.HIGHEST)
        + b2_ref[...])


def _encode(x, w1, b1, w2, b2):
    blk = 2000
    return pl.pallas_call(
        _enc_body,
        grid=(N // blk,),
        in_specs=[
            pl.BlockSpec((blk, D_IN), lambda i: (i, 0)),
            pl.BlockSpec((D_IN, H), lambda i: (0, 0)),
            pl.BlockSpec((1, H), lambda i: (0, 0)),
            pl.BlockSpec((H, H), lambda i: (0, 0)),
            pl.BlockSpec((1, H), lambda i: (0, 0)),
        ],
        out_specs=pl.BlockSpec((blk, H), lambda i: (i, 0)),
        out_shape=jax.ShapeDtypeStruct((N, H), jnp.float32),
    )(x, w1, b1.reshape(1, H), w2, b2.reshape(1, H))


def _ab_body(h_ref, w1_ref, b1_ref, o_ref):
    h = h_ref[...]
    w1 = w1_ref[...]
    wd = w1[0:H, :] - w1[H:2 * H, :]
    ws = w1[H:2 * H, :]
    a = jnp.dot(h, wd, preferred_element_type=jnp.float32,
                precision=lax.Precision.HIGHEST) + b1_ref[...]
    b = jnp.dot(h, ws, preferred_element_type=jnp.float32,
                precision=lax.Precision.HIGHEST)
    o_ref[...] = jnp.concatenate([a, b], axis=1)


def _node_tables(h, w1, b1):
    blk = 2000
    return pl.pallas_call(
        _ab_body,
        grid=(N // blk,),
        in_specs=[
            pl.BlockSpec((blk, H), lambda i: (i, 0)),
            pl.BlockSpec((2 * H, H), lambda i: (0, 0)),
            pl.BlockSpec((1, H), lambda i: (0, 0)),
        ],
        out_specs=pl.BlockSpec((blk, 2 * H), lambda i: (i, 0)),
        out_shape=jax.ShapeDtypeStruct((N, 2 * H), jnp.float32),
    )(h, w1, b1.reshape(1, H))


def _msg_body(t_ref, w2_ref, b2_ref, o_ref):
    t = t_ref[...]
    w2 = w2_ref[...]
    b2 = b2_ref[...]
    ma = jnp.maximum(
        jnp.dot(t[:, :H], w2, preferred_element_type=jnp.float32,
                precision=lax.Precision.HIGHEST) + b2, 0.0)
    mb = jnp.maximum(
        jnp.dot(t[:, H:], w2, preferred_element_type=jnp.float32,
                precision=lax.Precision.HIGHEST) + b2, 0.0)
    o_ref[...] = jnp.concatenate([ma, mb], axis=1)


def _messages(t, w2, b2):
    blk2 = 1000
    return pl.pallas_call(
        _msg_body,
        grid=(E // 2 // blk2,),
        in_specs=[
            pl.BlockSpec((blk2, 2 * H), lambda i: (i, 0)),
            pl.BlockSpec((H, H), lambda i: (0, 0)),
            pl.BlockSpec((1, H), lambda i: (0, 0)),
        ],
        out_specs=pl.BlockSpec((blk2, 2 * H), lambda i: (i, 0)),
        out_shape=jax.ShapeDtypeStruct((E // 2, 2 * H), jnp.float32),
    )(t, w2, b2.reshape(1, H))


def _gather_body(ei, ab_hbm, t_hbm, sidx, didx, bufs, bufd, buft,
                 semas, sembs, semt):
    wid = lax.axis_index("s") * NC + lax.axis_index("c")
    base = pl.multiple_of(wid * EPT, EPT)
    nch = EPT // GCH

    pltpu.sync_copy(ei.at[pl.ds(base, EPT)], sidx)
    pltpu.sync_copy(ei.at[pl.ds(E + base, EPT)], didx)

    def issue(ck, b):
        boff = pl.multiple_of(ck * GCH, GCH)
        pltpu.async_copy(ab_hbm.at[didx.at[pl.ds(boff, GCH)]], bufd[b],
                         semas[b])
        pltpu.async_copy(ab_hbm.at[sidx.at[pl.ds(boff, GCH)]], bufs[b],
                         sembs[b])

    def consume(ck, b, first):
        pltpu.make_async_copy(ab_hbm.at[didx.at[pl.ds(0, GCH)]], bufd[b],
                              semas[b]).wait()
        pltpu.make_async_copy(ab_hbm.at[sidx.at[pl.ds(0, GCH)]], bufs[b],
                              sembs[b]).wait()
        off2 = pl.multiple_of(base // 2 + ck * (GCH // 2), GCH // 2)
        if not first:
            pltpu.make_async_copy(buft, t_hbm.at[pl.ds(0, GCH // 2)],
                                  semt).wait()

        def row(k, c2):
            for c in range(H // 16):
                sa = pl.ds(c * 16, 16)
                sb = pl.ds(H + c * 16, 16)
                buft[k, sa] = jnp.maximum(
                    bufd[b][2 * k, sa] + bufs[b][2 * k, sb], 0.0)
                buft[k, sb] = jnp.maximum(
                    bufd[b][2 * k + 1, sa] + bufs[b][2 * k + 1, sb], 0.0)
            return c2

        lax.fori_loop(0, GCH // 2, row, 0, unroll=4)
        pltpu.async_copy(buft, t_hbm.at[pl.ds(off2, GCH // 2)], semt)

    issue(0, 0)
    issue(1, 1)
    consume(0, 0, True)

    def pair(j, carry):
        issue(2 * j + 2, 0)
        consume(2 * j + 1, 1, False)
        issue(2 * j + 3, 1)
        consume(2 * j + 2, 0, False)
        return carry

    lax.fori_loop(0, (nch - 3) // 2, pair, 0)
    issue(nch - 1, 0)
    consume(nch - 2, 1, False)
    consume(nch - 1, 0, False)
    pltpu.make_async_copy(buft, t_hbm.at[pl.ds(0, GCH // 2)], semt).wait()


def _gather_combine(edge_index, ab):
    mesh = plsc.VectorSubcoreMesh(core_axis_name="c", subcore_axis_name="s")
    f = functools.partial(
        pl.kernel,
        out_type=jax.ShapeDtypeStruct((E // 2, 2 * H), jnp.float32),
        mesh=mesh,
        scratch_types=[
            pltpu.VMEM((EPT,), jnp.int32),
            pltpu.VMEM((EPT,), jnp.int32),
            [pltpu.VMEM((GCH, 2 * H), jnp.float32)] * 2,
            [pltpu.VMEM((GCH, 2 * H), jnp.float32)] * 2,
            pltpu.VMEM((GCH // 2, 2 * H), jnp.float32),
            [pltpu.SemaphoreType.DMA] * 2,
            [pltpu.SemaphoreType.DMA] * 2,
            pltpu.SemaphoreType.DMA,
        ],
    )(_gather_body)
    return f(edge_index, ab)


def _route_body(ei, dl_hbm, rl_hbm, cnt_hbm, sbuf, dlist, rowlist, cbuf):
    wid = lax.axis_index("s") * NC + lax.axis_index("c")
    lo2 = wid * NPT2
    iota = lax.iota(jnp.int32, 16)

    pad_code = jnp.full((16,), (lo2 + NPT2) * 4, jnp.int32)
    zeros_i = jnp.zeros((16,), jnp.int32)

    def pre_lists(i, c2):
        dlist[pl.ds(i * 16, 16)] = pad_code
        rowlist[pl.ds(i * 16, 16)] = zeros_i
        return c2

    lax.fori_loop(0, (CAP + 16) // 16, pre_lists, 0, unroll=4)

    def scan_chunk(ck, ptrv):
        pltpu.sync_copy(
            ei.at[pl.ds(pl.multiple_of(E + ck * RCH, RCH), RCH)], sbuf)

        def scan_v(v, ptrv):
            d = sbuf[pl.ds(v * 16, 16)]
            pr = d >> 1
            mask = (pr >= lo2) & (pr < lo2 + NPT2)
            eloc = v * 16 + iota
            keys = jnp.where(mask, 0, 16) + iota
            packed = jnp.where(mask, (eloc << 9) | (d - 2 * lo2),
                               2 * NPT2)
            _, sp = plsc.sort_key_val(keys, packed)
            eid2 = ck * RCH + (sp >> 9)
            dcode = ((sp & 511) + 2 * lo2) * 2 + (eid2 & 1)
            offs = ptrv + iota
            plsc.store_scatter(dlist, [offs], dcode)
            plsc.store_scatter(rowlist, [offs], eid2 >> 1)
            return ptrv + plsc.all_reduce_population_count(mask)

        return lax.fori_loop(0, RCH // 16, scan_v, ptrv, unroll=2)

    cntv = lax.fori_loop(0, E // RCH, scan_chunk,
                         jnp.zeros((16,), jnp.int32))

    cbuf[pl.ds(0, 16)] = cntv
    pltpu.sync_copy(dlist.at[pl.ds(0, CAP)],
                    dl_hbm.at[pl.ds(pl.multiple_of(wid * CAP, CAP), CAP)])
    pltpu.sync_copy(rowlist.at[pl.ds(0, CAP)],
                    rl_hbm.at[pl.ds(pl.multiple_of(wid * CAP, CAP), CAP)])
    pltpu.sync_copy(cbuf, cnt_hbm.at[pl.ds(pl.multiple_of(wid * 16, 16), 16)])


def _route(edge_index):
    mesh = plsc.VectorSubcoreMesh(core_axis_name="c", subcore_axis_name="s")
    f = functools.partial(
        pl.kernel,
        out_type=[
            jax.ShapeDtypeStruct((NW * CAP,), jnp.int32),
            jax.ShapeDtypeStruct((NW * CAP,), jnp.int32),
            jax.ShapeDtypeStruct((NW * 16,), jnp.int32),
        ],
        mesh=mesh,
        compiler_params=pltpu.CompilerParams(needs_layout_passes=False),
        scratch_types=[
            pltpu.VMEM((RCH,), jnp.int32),
            pltpu.VMEM((CAP + 16,), jnp.int32),
            pltpu.VMEM((CAP + 16,), jnp.int32),
            pltpu.VMEM((16,), jnp.int32),
        ],
    )(_route_body)
    return f(edge_index)


def _scatter_body(dl_hbm, rl_hbm, cnt_hbm, m_hbm, out_hbm, dlist, rowlist,
                  mbufs, acc, cbuf, sems):
    wid = lax.axis_index("s") * NC + lax.axis_index("c")
    lo2 = wid * NPT2
    zeros_f = jnp.zeros((16,), jnp.float32)

    pltpu.sync_copy(cnt_hbm.at[pl.ds(pl.multiple_of(wid * 16, 16), 16)], cbuf)
    pltpu.sync_copy(dl_hbm.at[pl.ds(pl.multiple_of(wid * CAP, CAP), CAP)],
                    dlist)
    pltpu.sync_copy(rl_hbm.at[pl.ds(pl.multiple_of(wid * CAP, CAP), CAP)],
                    rowlist)

    def pre_acc(i, c2):
        for c in range(2 * H // 16):
            acc[i, pl.ds(c * 16, 16)] = zeros_f
        return c2

    lax.fori_loop(0, NPT2 + 1, pre_acc, 0, unroll=4)

    cnt = cbuf[pl.ds(0, 16)][0]
    trips = (cnt + (BCH - 1)) // BCH

    def issue(ck, b):
        @pl.when(ck < trips)
        def _():
            boff = pl.multiple_of(ck * BCH, BCH)
            pltpu.async_copy(m_hbm.at[rowlist.at[pl.ds(boff, BCH)]],
                             mbufs[b], sems[b])

    def process(ck, b):
        @pl.when(ck < trips)
        def _():
            pltpu.make_async_copy(m_hbm.at[rowlist.at[pl.ds(0, BCH)]],
                                  mbufs[b], sems[b]).wait()

            def grp(g, c3):
                base16 = ck * BCH + g * 16
                dv = dlist[pl.ds(base16, 16)]
                for e in range(16):
                    dc = dv[e]
                    d = dc >> 1
                    r = (d >> 1) - lo2
                    ab_ = (d & 1) * H
                    mb_ = (dc & 1) * H
                    j = g * 16 + e
                    for c in range(H // 16):
                        mv = mbufs[b][j, pl.ds(mb_ + c * 16, 16)]
                        av = acc[r, pl.ds(ab_ + c * 16, 16)]
                        acc[r, pl.ds(ab_ + c * 16, 16)] = jnp.maximum(av, mv)
                return c3

            lax.fori_loop(0, BCH // 16, grp, 0, unroll=2)

    issue(0, 0)

    def pairb(j, c2):
        issue(2 * j + 1, 1)
        process(2 * j, 0)
        issue(2 * j + 2, 0)
        process(2 * j + 1, 1)
        return c2

    lax.fori_loop(0, (trips + 1) // 2, pairb, 0)

    pltpu.sync_copy(acc.at[pl.ds(0, NPT2)],
                    out_hbm.at[pl.ds(pl.multiple_of(lo2, NPT2), NPT2)])


def _segment_max(dl, rl, cnts, m):
    mesh = plsc.VectorSubcoreMesh(core_axis_name="c", subcore_axis_name="s")
    f = functools.partial(
        pl.kernel,
        out_type=jax.ShapeDtypeStruct((NPAD2, 2 * H), jnp.float32),
        mesh=mesh,
        compiler_params=pltpu.CompilerParams(needs_layout_passes=False),
        scratch_types=[
            pltpu.VMEM((CAP,), jnp.int32),
            pltpu.VMEM((CAP,), jnp.int32),
            [pltpu.VMEM((BCH, 2 * H), jnp.float32)] * 2,
            pltpu.VMEM((NPT2 + 1, 2 * H), jnp.float32),
            pltpu.VMEM((16,), jnp.int32),
            [pltpu.SemaphoreType.DMA] * 2,
        ],
    )(_scatter_body)
    return f(dl, rl, cnts, m)


def kernel(x, edge_index, nW1, nb1, nW2, nb2,
           c0W1, c0b1, c0W2, c0b2,
           c1W1, c1b1, c1W2, c1b2,
           c2W1, c2b1, c2W2, c2b2):
    ei_flat = edge_index.reshape(2 * E)
    h = _encode(x, nW1, nb1, nW2, nb2)
    dl, rl, cnts = _route(ei_flat)
    for (w1, b1, w2, b2) in ((c0W1, c0b1, c0W2, c0b2),
                             (c1W1, c1b1, c1W2, c1b2),
                             (c2W1, c2b1, c2W2, c2b2)):
        ab = _node_tables(h, w1, b1)
        t = _gather_combine(ei_flat, ab)
        m = _messages(t, w2, b2)
        h = _segment_max(dl, rl, cnts, m).reshape(2 * NPAD2, H)[:N]
    return h


# speedup vs baseline: 2.5259x; 1.0004x over previous
import functools

import jax
import jax.numpy as jnp
from jax import lax
from jax.experimental import pallas as pl
from jax.experimental.pallas import tpu as pltpu
from jax.experimental.pallas import tpu_sc as plsc

N = 10000
E = 320000
D_IN = 128
H = 64

NC = 2
NS = 16
NW = NC * NS

NPT2 = 160
NPAD2 = NW * NPT2
EPT = E // NW
GCH = 80
RCH = 8000
CAP = 16384
BCH = 256


def _enc_body(x_ref, w1_ref, b1_ref, w2_ref, b2_ref, o_ref):
    h1 = jnp.maximum(
        jnp.dot(x_ref[...], w1_ref[...], preferred_element_type=jnp.float32,
                precision=lax.Precision.HIGHEST)
        + b1_ref[...], 0.0)
    o_ref[...] = (
        jnp.dot(h1, w2_ref[...], preferred_element_type=jnp.float32,
                precision=lax.Precision.HIGHEST)
        + b2_ref[...])


def _encode(x, w1, b1, w2, b2):
    blk = 2000
    return pl.pallas_call(
        _enc_body,
        grid=(N // blk,),
        in_specs=[
            pl.BlockSpec((blk, D_IN), lambda i: (i, 0)),
            pl.BlockSpec((D_IN, H), lambda i: (0, 0)),
            pl.BlockSpec((1, H), lambda i: (0, 0)),
            pl.BlockSpec((H, H), lambda i: (0, 0)),
            pl.BlockSpec((1, H), lambda i: (0, 0)),
        ],
        out_specs=pl.BlockSpec((blk, H), lambda i: (i, 0)),
        out_shape=jax.ShapeDtypeStruct((N, H), jnp.float32),
    )(x, w1, b1.reshape(1, H), w2, b2.reshape(1, H))


def _ab_body(h_ref, w1_ref, b1_ref, o_ref):
    h = h_ref[...]
    w1 = w1_ref[...]
    wd = w1[0:H, :] - w1[H:2 * H, :]
    ws = w1[H:2 * H, :]
    a = jnp.dot(h, wd, preferred_element_type=jnp.float32,
                precision=lax.Precision.HIGHEST) + b1_ref[...]
    b = jnp.dot(h, ws, preferred_element_type=jnp.float32,
                precision=lax.Precision.HIGHEST)
    o_ref[...] = jnp.concatenate([a, b], axis=1)


def _node_tables(h, w1, b1):
    blk = 2000
    return pl.pallas_call(
        _ab_body,
        grid=(N // blk,),
        in_specs=[
            pl.BlockSpec((blk, H), lambda i: (i, 0)),
            pl.BlockSpec((2 * H, H), lambda i: (0, 0)),
            pl.BlockSpec((1, H), lambda i: (0, 0)),
        ],
        out_specs=pl.BlockSpec((blk, 2 * H), lambda i: (i, 0)),
        out_shape=jax.ShapeDtypeStruct((N, 2 * H), jnp.float32),
    )(h, w1, b1.reshape(1, H))


def _msg_body(t_ref, w2_ref, b2_ref, o_ref):
    t = t_ref[...]
    w2 = w2_ref[...]
    b2 = b2_ref[...]
    ma = jnp.maximum(
        jnp.dot(t[:, :H], w2, preferred_element_type=jnp.float32,
                precision=lax.Precision.HIGHEST) + b2, 0.0)
    mb = jnp.maximum(
        jnp.dot(t[:, H:], w2, preferred_element_type=jnp.float32,
                precision=lax.Precision.HIGHEST) + b2, 0.0)
    o_ref[...] = jnp.concatenate([ma, mb], axis=1)


def _messages(t, w2, b2):
    blk2 = 1000
    return pl.pallas_call(
        _msg_body,
        grid=(E // 2 // blk2,),
        in_specs=[
            pl.BlockSpec((blk2, 2 * H), lambda i: (i, 0)),
            pl.BlockSpec((H, H), lambda i: (0, 0)),
            pl.BlockSpec((1, H), lambda i: (0, 0)),
        ],
        out_specs=pl.BlockSpec((blk2, 2 * H), lambda i: (i, 0)),
        out_shape=jax.ShapeDtypeStruct((E // 2, 2 * H), jnp.float32),
    )(t, w2, b2.reshape(1, H))


def _gather_body(ei, ab_hbm, t_hbm, sidx, didx, bufs, bufd, buft,
                 semas, sembs, semt):
    wid = lax.axis_index("s") * NC + lax.axis_index("c")
    base = pl.multiple_of(wid * EPT, EPT)
    nch = EPT // GCH

    pltpu.sync_copy(ei.at[pl.ds(base, EPT)], sidx)
    pltpu.sync_copy(ei.at[pl.ds(E + base, EPT)], didx)

    def issue(ck, b):
        boff = pl.multiple_of(ck * GCH, GCH)
        pltpu.async_copy(ab_hbm.at[didx.at[pl.ds(boff, GCH)]], bufd[b],
                         semas[b])
        pltpu.async_copy(ab_hbm.at[sidx.at[pl.ds(boff, GCH)]], bufs[b],
                         sembs[b])

    def consume(ck, b, first):
        pltpu.make_async_copy(ab_hbm.at[didx.at[pl.ds(0, GCH)]], bufd[b],
                              semas[b]).wait()
        pltpu.make_async_copy(ab_hbm.at[sidx.at[pl.ds(0, GCH)]], bufs[b],
                              sembs[b]).wait()
        off2 = pl.multiple_of(base // 2 + ck * (GCH // 2), GCH // 2)
        if not first:
            pltpu.make_async_copy(buft, t_hbm.at[pl.ds(0, GCH // 2)],
                                  semt).wait()

        def row(k, c2):
            for c in range(H // 16):
                sa = pl.ds(c * 16, 16)
                sb = pl.ds(H + c * 16, 16)
                buft[k, sa] = jnp.maximum(
                    bufd[b][2 * k, sa] + bufs[b][2 * k, sb], 0.0)
                buft[k, sb] = jnp.maximum(
                    bufd[b][2 * k + 1, sa] + bufs[b][2 * k + 1, sb], 0.0)
            return c2

        lax.fori_loop(0, GCH // 2, row, 0, unroll=4)
        pltpu.async_copy(buft, t_hbm.at[pl.ds(off2, GCH // 2)], semt)

    issue(0, 0)
    issue(1, 1)
    issue(2, 2)
    consume(0, 0, True)

    def quad(j, carry):
        for q in range(4):
            k = 4 * j + q + 1

            @pl.when(k + 2 < nch)
            def _():
                issue(k + 2, (q + 3) % 4)

            consume(k, (q + 1) % 4, False)
        return carry

    lax.fori_loop(0, (nch - 1) // 4, quad, 0)
    pltpu.make_async_copy(buft, t_hbm.at[pl.ds(0, GCH // 2)], semt).wait()


def _gather_combine(edge_index, ab):
    mesh = plsc.VectorSubcoreMesh(core_axis_name="c", subcore_axis_name="s")
    f = functools.partial(
        pl.kernel,
        out_type=jax.ShapeDtypeStruct((E // 2, 2 * H), jnp.float32),
        mesh=mesh,
        scratch_types=[
            pltpu.VMEM((EPT,), jnp.int32),
            pltpu.VMEM((EPT,), jnp.int32),
            [pltpu.VMEM((GCH, 2 * H), jnp.float32)] * 4,
            [pltpu.VMEM((GCH, 2 * H), jnp.float32)] * 4,
            pltpu.VMEM((GCH // 2, 2 * H), jnp.float32),
            [pltpu.SemaphoreType.DMA] * 4,
            [pltpu.SemaphoreType.DMA] * 4,
            pltpu.SemaphoreType.DMA,
        ],
    )(_gather_body)
    return f(edge_index, ab)


def _route_body(ei, dl_hbm, rl_hbm, cnt_hbm, sbuf, dlist, rowlist, cbuf):
    wid = lax.axis_index("s") * NC + lax.axis_index("c")
    lo2 = wid * NPT2
    iota = lax.iota(jnp.int32, 16)

    pad_code = jnp.full((16,), (lo2 + NPT2) * 4, jnp.int32)
    zeros_i = jnp.zeros((16,), jnp.int32)

    def pre_lists(i, c2):
        dlist[pl.ds(i * 16, 16)] = pad_code
        rowlist[pl.ds(i * 16, 16)] = zeros_i
        return c2

    lax.fori_loop(0, (CAP + 16) // 16, pre_lists, 0, unroll=4)

    def scan_chunk(ck, ptrv):
        pltpu.sync_copy(
            ei.at[pl.ds(pl.multiple_of(E + ck * RCH, RCH), RCH)], sbuf)

        def scan_v(v, ptrv):
            d = sbuf[pl.ds(v * 16, 16)]
            pr = d >> 1
            mask = (pr >= lo2) & (pr < lo2 + NPT2)
            eloc = v * 16 + iota
            keys = jnp.where(mask, 0, 16) + iota
            packed = jnp.where(mask, (eloc << 9) | (d - 2 * lo2),
                               2 * NPT2)
            _, sp = plsc.sort_key_val(keys, packed)
            eid2 = ck * RCH + (sp >> 9)
            dcode = ((sp & 511) + 2 * lo2) * 2 + (eid2 & 1)
            offs = ptrv + iota
            plsc.store_scatter(dlist, [offs], dcode)
            plsc.store_scatter(rowlist, [offs], eid2 >> 1)
            return ptrv + plsc.all_reduce_population_count(mask)

        return lax.fori_loop(0, RCH // 16, scan_v, ptrv, unroll=2)

    cntv = lax.fori_loop(0, E // RCH, scan_chunk,
                         jnp.zeros((16,), jnp.int32))

    cbuf[pl.ds(0, 16)] = cntv
    pltpu.sync_copy(dlist.at[pl.ds(0, CAP)],
                    dl_hbm.at[pl.ds(pl.multiple_of(wid * CAP, CAP), CAP)])
    pltpu.sync_copy(rowlist.at[pl.ds(0, CAP)],
                    rl_hbm.at[pl.ds(pl.multiple_of(wid * CAP, CAP), CAP)])
    pltpu.sync_copy(cbuf, cnt_hbm.at[pl.ds(pl.multiple_of(wid * 16, 16), 16)])


def _route(edge_index):
    mesh = plsc.VectorSubcoreMesh(core_axis_name="c", subcore_axis_name="s")
    f = functools.partial(
        pl.kernel,
        out_type=[
            jax.ShapeDtypeStruct((NW * CAP,), jnp.int32),
            jax.ShapeDtypeStruct((NW * CAP,), jnp.int32),
            jax.ShapeDtypeStruct((NW * 16,), jnp.int32),
        ],
        mesh=mesh,
        compiler_params=pltpu.CompilerParams(needs_layout_passes=False),
        scratch_types=[
            pltpu.VMEM((RCH,), jnp.int32),
            pltpu.VMEM((CAP + 16,), jnp.int32),
            pltpu.VMEM((CAP + 16,), jnp.int32),
            pltpu.VMEM((16,), jnp.int32),
        ],
    )(_route_body)
    return f(edge_index)


def _scatter_body(dl_hbm, rl_hbm, cnt_hbm, m_hbm, out_hbm, dlist, rowlist,
                  mbufs, acc, cbuf, sems):
    wid = lax.axis_index("s") * NC + lax.axis_index("c")
    lo2 = wid * NPT2
    zeros_f = jnp.zeros((16,), jnp.float32)

    pltpu.sync_copy(cnt_hbm.at[pl.ds(pl.multiple_of(wid * 16, 16), 16)], cbuf)
    pltpu.sync_copy(dl_hbm.at[pl.ds(pl.multiple_of(wid * CAP, CAP), CAP)],
                    dlist)
    pltpu.sync_copy(rl_hbm.at[pl.ds(pl.multiple_of(wid * CAP, CAP), CAP)],
                    rowlist)

    def pre_acc(i, c2):
        for c in range(2 * H // 16):
            acc[i, pl.ds(c * 16, 16)] = zeros_f
        return c2

    lax.fori_loop(0, NPT2 + 1, pre_acc, 0, unroll=4)

    cnt = cbuf[pl.ds(0, 16)][0]
    trips = (cnt + (BCH - 1)) // BCH

    def issue(ck, b):
        @pl.when(ck < trips)
        def _():
            boff = pl.multiple_of(ck * BCH, BCH)
            pltpu.async_copy(m_hbm.at[rowlist.at[pl.ds(boff, BCH)]],
                             mbufs[b], sems[b])

    def process(ck, b):
        @pl.when(ck < trips)
        def _():
            pltpu.make_async_copy(m_hbm.at[rowlist.at[pl.ds(0, BCH)]],
                                  mbufs[b], sems[b]).wait()

            def grp(g, c3):
                base16 = ck * BCH + g * 16
                dv = dlist[pl.ds(base16, 16)]
                for e in range(16):
                    dc = dv[e]
                    d = dc >> 1
                    r = (d >> 1) - lo2
                    ab_ = (d & 1) * H
                    mb_ = (dc & 1) * H
                    j = g * 16 + e
                    for c in range(H // 16):
                        mv = mbufs[b][j, pl.ds(mb_ + c * 16, 16)]
                        av = acc[r, pl.ds(ab_ + c * 16, 16)]
                        acc[r, pl.ds(ab_ + c * 16, 16)] = jnp.maximum(av, mv)
                return c3

            lax.fori_loop(0, BCH // 16, grp, 0, unroll=2)

    issue(0, 0)

    def pairb(j, c2):
        issue(2 * j + 1, 1)
        process(2 * j, 0)
        issue(2 * j + 2, 0)
        process(2 * j + 1, 1)
        return c2

    lax.fori_loop(0, (trips + 1) // 2, pairb, 0)

    pltpu.sync_copy(acc.at[pl.ds(0, NPT2)],
                    out_hbm.at[pl.ds(pl.multiple_of(lo2, NPT2), NPT2)])


def _segment_max(dl, rl, cnts, m):
    mesh = plsc.VectorSubcoreMesh(core_axis_name="c", subcore_axis_name="s")
    f = functools.partial(
        pl.kernel,
        out_type=jax.ShapeDtypeStruct((NPAD2, 2 * H), jnp.float32),
        mesh=mesh,
        compiler_params=pltpu.CompilerParams(needs_layout_passes=False),
        scratch_types=[
            pltpu.VMEM((CAP,), jnp.int32),
            pltpu.VMEM((CAP,), jnp.int32),
            [pltpu.VMEM((BCH, 2 * H), jnp.float32)] * 2,
            pltpu.VMEM((NPT2 + 1, 2 * H), jnp.float32),
            pltpu.VMEM((16,), jnp.int32),
            [pltpu.SemaphoreType.DMA] * 2,
        ],
    )(_scatter_body)
    return f(dl, rl, cnts, m)


def kernel(x, edge_index, nW1, nb1, nW2, nb2,
           c0W1, c0b1, c0W2, c0b2,
           c1W1, c1b1, c1W2, c1b2,
           c2W1, c2b1, c2W2, c2b2):
    ei_flat = edge_index.reshape(2 * E)
    h = _encode(x, nW1, nb1, nW2, nb2)
    dl, rl, cnts = _route(ei_flat)
    for (w1, b1, w2, b2) in ((c0W1, c0b1, c0W2, c0b2),
                             (c1W1, c1b1, c1W2, c1b2),
                             (c2W1, c2b1, c2W2, c2b2)):
        ab = _node_tables(h, w1, b1)
        t = _gather_combine(ei_flat, ab)
        m = _messages(t, w2, b2)
        h = _segment_max(dl, rl, cnts, m).reshape(2 * NPAD2, H)[:N]
    return h
